# Initial kernel scaffold; baseline (speedup 1.0000x reference)
import jax
import jax.numpy as jnp
from jax.experimental import pallas as pl


def kernel(x, W1, W2, ln1_w, ln2_w):
    raise NotImplementedError("write your pallas kernel here")


# speedup vs baseline: 6.3198x; 6.3198x over previous
import functools

import jax
import jax.numpy as jnp
from jax.experimental import pallas as pl
from jax.experimental.pallas import tpu as pltpu

_HEADS = 6
_EXPERTS = 512
_TOPK = 8
_HIDDEN = 2048
_FLAT = _HEADS * _EXPERTS
_NEG = -10000000000.0
_EPS = 1e-5


def _router_block(lnw_ref, x_ref, w1_ref, w2_ref, o1_ref, o2_ref):
    bT = x_ref.shape[0]
    x = x_ref[...]
    iota = jax.lax.broadcasted_iota(jnp.int32, (bT, _EXPERTS), 1)
    for mi, (w_ref, o_ref) in enumerate(((w1_ref, o1_ref), (w2_ref, o2_ref))):
        gz = jnp.dot(x, w_ref[...], preferred_element_type=jnp.float32)
        heads = [gz[:, h * _EXPERTS:(h + 1) * _EXPERTS] for h in range(_HEADS)]
        mean = heads[0]
        for h in range(1, _HEADS):
            mean = mean + heads[h]
        mean = mean * (1.0 / _HEADS)
        var = (heads[0] - mean) ** 2
        for h in range(1, _HEADS):
            var = var + (heads[h] - mean) ** 2
        var = var * (1.0 / _HEADS)
        inv = 1.0 / jnp.sqrt(var + _EPS)
        for h in range(_HEADS):
            gn = (heads[h] - mean) * inv * lnw_ref[mi, h]
            work = gn
            for _ in range(_TOPK - 1):
                m = jnp.max(work, axis=1, keepdims=True)
                is_max = work == m
                first = jnp.min(jnp.where(is_max, iota, _EXPERTS),
                                axis=1, keepdims=True)
                work = jnp.where(iota == first, -jnp.inf, work)
            thr = jnp.max(work, axis=1, keepdims=True)
            masked = jnp.where(gn >= thr, heads[h], _NEG)
            mx = jnp.max(masked, axis=1, keepdims=True)
            e = jnp.exp(masked - mx)
            o_ref[:, h * _EXPERTS:(h + 1) * _EXPERTS] = e / jnp.sum(
                e, axis=1, keepdims=True)


@functools.partial(jax.jit, static_argnames=("block_t",))
def _router(x, W1, W2, lnw, block_t=256):
    T = x.shape[0]
    xb = x.astype(jnp.bfloat16)
    w1t = W1.T.astype(jnp.bfloat16)
    w2t = W2.T.astype(jnp.bfloat16)
    grid = (T // block_t,)
    out1, out2 = pl.pallas_call(
        _router_block,
        grid=grid,
        in_specs=[
            pl.BlockSpec(memory_space=pltpu.SMEM),
            pl.BlockSpec((block_t, _HIDDEN), lambda i: (i, 0)),
            pl.BlockSpec((_HIDDEN, _FLAT), lambda i: (0, 0)),
            pl.BlockSpec((_HIDDEN, _FLAT), lambda i: (0, 0)),
        ],
        out_specs=[
            pl.BlockSpec((block_t, _FLAT), lambda i: (i, 0)),
            pl.BlockSpec((block_t, _FLAT), lambda i: (i, 0)),
        ],
        out_shape=[
            jax.ShapeDtypeStruct((T, _FLAT), jnp.float32),
            jax.ShapeDtypeStruct((T, _FLAT), jnp.float32),
        ],
        compiler_params=pltpu.CompilerParams(
            dimension_semantics=("arbitrary",),
        ),
    )(lnw, xb, w1t, w2t)
    return out1, out2


def kernel(x, W1, W2, ln1_w, ln2_w):
    T = x.shape[0]
    lnw = jnp.stack([ln1_w, ln2_w])
    o1, o2 = _router(x, W1, W2, lnw)
    return (o1.reshape(T, _HEADS, _EXPERTS), o2.reshape(T, _HEADS, _EXPERTS))


# speedup vs baseline: 9.5471x; 1.5107x over previous
import functools

import jax
import jax.numpy as jnp
from jax.experimental import pallas as pl
from jax.experimental.pallas import tpu as pltpu

_HEADS = 6
_EXPERTS = 512
_TOPK = 8
_HIDDEN = 2048
_FLAT = _HEADS * _EXPERTS
_NEG = -10000000000.0
_EPS = 1e-5


def _router_block(lnw_ref, xt_ref, w1_ref, w2_ref, o1_ref, o2_ref):
    xt = xt_ref[...]
    for mi, (w_ref, o_ref) in enumerate(((w1_ref, o1_ref), (w2_ref, o2_ref))):
        gz = jnp.dot(w_ref[...], xt, preferred_element_type=jnp.float32)
        heads = [gz[h * _EXPERTS:(h + 1) * _EXPERTS, :] for h in range(_HEADS)]
        mean = heads[0]
        for h in range(1, _HEADS):
            mean = mean + heads[h]
        mean = mean * (1.0 / _HEADS)
        var = (heads[0] - mean) ** 2
        for h in range(1, _HEADS):
            var = var + (heads[h] - mean) ** 2
        var = var * (1.0 / _HEADS)
        inv = 1.0 / jnp.sqrt(var + _EPS)
        for h in range(_HEADS):
            gn = (heads[h] - mean) * inv * lnw_ref[mi, h]
            work = gn
            for _ in range(_TOPK - 1):
                m = jnp.max(work, axis=0, keepdims=True)
                work = jnp.where(work == m, -jnp.inf, work)
            thr = jnp.max(work, axis=0, keepdims=True)
            masked = jnp.where(gn >= thr, heads[h], _NEG)
            mx = jnp.max(masked, axis=0, keepdims=True)
            e = jnp.exp(masked - mx)
            o_ref[h * _EXPERTS:(h + 1) * _EXPERTS, :] = e / jnp.sum(
                e, axis=0, keepdims=True)


@functools.partial(jax.jit, static_argnames=("block_t",))
def _router(x, W1, W2, lnw, block_t=256):
    T = x.shape[0]
    xt = x.T.astype(jnp.bfloat16)
    w1 = W1.astype(jnp.bfloat16)
    w2 = W2.astype(jnp.bfloat16)
    grid = (T // block_t,)
    out1, out2 = pl.pallas_call(
        _router_block,
        grid=grid,
        in_specs=[
            pl.BlockSpec(memory_space=pltpu.SMEM),
            pl.BlockSpec((_HIDDEN, block_t), lambda i: (0, i)),
            pl.BlockSpec((_FLAT, _HIDDEN), lambda i: (0, 0)),
            pl.BlockSpec((_FLAT, _HIDDEN), lambda i: (0, 0)),
        ],
        out_specs=[
            pl.BlockSpec((_FLAT, block_t), lambda i: (0, i)),
            pl.BlockSpec((_FLAT, block_t), lambda i: (0, i)),
        ],
        out_shape=[
            jax.ShapeDtypeStruct((_FLAT, T), jnp.float32),
            jax.ShapeDtypeStruct((_FLAT, T), jnp.float32),
        ],
        compiler_params=pltpu.CompilerParams(
            dimension_semantics=("arbitrary",),
        ),
    )(lnw, xt, w1, w2)
    return out1, out2


def kernel(x, W1, W2, ln1_w, ln2_w):
    T = x.shape[0]
    lnw = jnp.stack([ln1_w, ln2_w])
    o1t, o2t = _router(x, W1, W2, lnw)
    return (o1t.T.reshape(T, _HEADS, _EXPERTS),
            o2t.T.reshape(T, _HEADS, _EXPERTS))
